# Initial kernel scaffold; baseline (speedup 1.0000x reference)
import jax
import jax.numpy as jnp
from jax.experimental import pallas as pl


def kernel(weights, bias, input_0, input_1, input_2, input_3, input_4, input_5, input_6, input_7, input_8, input_9, input_10, input_11, input_12, input_13, input_14, input_15, input_16):
    raise NotImplementedError("write your pallas kernel here")


# speedup vs baseline: 8.7386x; 8.7386x over previous
import jax
import jax.numpy as jnp
from jax.experimental import pallas as pl
from jax.experimental.pallas import tpu as pltpu

K = 17
N = 50000
IC = 32
OC = 32
B = 512


def _body(w_ref, b_ref, *refs):
    in_refs = refs[:K]
    out_ref = refs[K]
    tail_ref = refs[K + 1]

    i = pl.program_id(0)
    m = jax.lax.broadcasted_iota(jnp.int32, (B, 1), 0) + i * B

    acc = None
    mask_cols = []
    for k in range(K):
        cur = in_refs[k][...]
        tail = tail_ref[k]
        z = jnp.concatenate([tail, cur], axis=0)
        shifted = jax.lax.slice(z, (K - 1 - k, 0), (K - 1 - k + B, IC))
        valid = jnp.logical_and(m >= k, m <= (N - 1) + k)
        shifted = jnp.where(valid, shifted, 0.0)
        part = jax.lax.dot_general(
            shifted, w_ref[k], (((1,), (1,)), ((), ())),
            preferred_element_type=jnp.float32)
        acc = part if acc is None else acc + part
        mask_cols.append(valid.astype(jnp.float32))
        tail_ref[k] = cur[B - (K - 1):, :]

    maskf = jnp.concatenate(mask_cols, axis=1)
    acc = acc + jax.lax.dot_general(
        maskf, b_ref[...], (((1,), (0,)), ((), ())),
        preferred_element_type=jnp.float32)
    out_ref[...] = acc


def kernel(weights, bias, input_0, input_1, input_2, input_3, input_4,
           input_5, input_6, input_7, input_8, input_9, input_10, input_11,
           input_12, input_13, input_14, input_15, input_16):
    ins = (input_0, input_1, input_2, input_3, input_4, input_5, input_6,
           input_7, input_8, input_9, input_10, input_11, input_12, input_13,
           input_14, input_15, input_16)
    n_out = N + K - 1
    grid = (pl.cdiv(n_out, B),)
    return pl.pallas_call(
        _body,
        grid=grid,
        in_specs=[
            pl.BlockSpec((K, OC, IC), lambda i: (0, 0, 0)),
            pl.BlockSpec((K, OC), lambda i: (0, 0)),
        ] + [pl.BlockSpec((B, IC), lambda i: (i, 0))] * K,
        out_specs=pl.BlockSpec((B, OC), lambda i: (i, 0)),
        out_shape=jax.ShapeDtypeStruct((n_out, OC), jnp.float32),
        scratch_shapes=[pltpu.VMEM((K, K - 1, IC), jnp.float32)],
        compiler_params=pltpu.CompilerParams(
            dimension_semantics=("arbitrary",)),
    )(weights, bias, *ins)


# speedup vs baseline: 9.1928x; 1.0520x over previous
import jax
import jax.numpy as jnp
from jax.experimental import pallas as pl
from jax.experimental.pallas import tpu as pltpu

K = 17
N = 50000
IC = 32
OC = 32
B = 1024


def _body(w_ref, b_ref, *refs):
    in_refs = refs[:K]
    out_ref = refs[K]
    tail_ref = refs[K + 1]

    i = pl.program_id(0)
    m = jax.lax.broadcasted_iota(jnp.int32, (B, 1), 0) + i * B

    acc = None
    mask_cols = []
    for k in range(K):
        cur = in_refs[k][...]
        tail = tail_ref[k]
        z = jnp.concatenate([tail, cur], axis=0)
        shifted = jax.lax.slice(z, (K - 1 - k, 0), (K - 1 - k + B, IC))
        valid = jnp.logical_and(m >= k, m <= (N - 1) + k)
        shifted = jnp.where(valid, shifted, 0.0)
        part = jax.lax.dot_general(
            shifted, w_ref[k], (((1,), (1,)), ((), ())),
            preferred_element_type=jnp.float32)
        acc = part if acc is None else acc + part
        mask_cols.append(valid.astype(jnp.float32))
        tail_ref[k] = cur[B - (K - 1):, :]

    maskf = jnp.concatenate(mask_cols, axis=1)
    acc = acc + jax.lax.dot_general(
        maskf, b_ref[...], (((1,), (0,)), ((), ())),
        preferred_element_type=jnp.float32)
    out_ref[...] = acc


def kernel(weights, bias, input_0, input_1, input_2, input_3, input_4,
           input_5, input_6, input_7, input_8, input_9, input_10, input_11,
           input_12, input_13, input_14, input_15, input_16):
    ins = (input_0, input_1, input_2, input_3, input_4, input_5, input_6,
           input_7, input_8, input_9, input_10, input_11, input_12, input_13,
           input_14, input_15, input_16)
    n_out = N + K - 1
    grid = (pl.cdiv(n_out, B),)
    return pl.pallas_call(
        _body,
        grid=grid,
        in_specs=[
            pl.BlockSpec((K, OC, IC), lambda i: (0, 0, 0)),
            pl.BlockSpec((K, OC), lambda i: (0, 0)),
        ] + [pl.BlockSpec((B, IC), lambda i: (i, 0))] * K,
        out_specs=pl.BlockSpec((B, OC), lambda i: (i, 0)),
        out_shape=jax.ShapeDtypeStruct((n_out, OC), jnp.float32),
        scratch_shapes=[pltpu.VMEM((K, K - 1, IC), jnp.float32)],
        compiler_params=pltpu.CompilerParams(
            dimension_semantics=("arbitrary",)),
    )(weights, bias, *ins)


# speedup vs baseline: 9.5987x; 1.0442x over previous
import jax
import jax.numpy as jnp
from jax.experimental import pallas as pl
from jax.experimental.pallas import tpu as pltpu

K = 17
N = 50000
IC = 32
OC = 32
B = 1024


def _body(w_ref, b_ref, *refs):
    in_refs = refs[:K]
    out_ref = refs[K]
    tail_ref = refs[K + 1]

    i = pl.program_id(0)
    num_steps = pl.num_programs(0)

    curs = [in_refs[k][...] for k in range(K)]
    zs = [jnp.concatenate([tail_ref[k], curs[k]], axis=0) for k in range(K)]
    shifted = [jax.lax.slice(zs[k], (K - 1 - k, 0), (K - 1 - k + B, IC))
               for k in range(K)]

    def matsum(parts):
        acc = None
        for k in range(K):
            p = jax.lax.dot_general(parts[k], w_ref[k], (((1,), (1,)), ((), ())),
                                    preferred_element_type=jnp.float32)
            acc = p if acc is None else acc + p
        return acc

    @pl.when(jnp.logical_and(i > 0, i < num_steps - 1))
    def _fast():
        acc = matsum(shifted)
        out_ref[...] = acc + jnp.sum(b_ref[...], axis=0, keepdims=True)

    @pl.when(jnp.logical_or(i == 0, i == num_steps - 1))
    def _edge():
        m = jax.lax.broadcasted_iota(jnp.int32, (B, 1), 0) + i * B
        masked = []
        mask_cols = []
        for k in range(K):
            valid = jnp.logical_and(m >= k, m <= (N - 1) + k)
            masked.append(jnp.where(valid, shifted[k], 0.0))
            mask_cols.append(valid.astype(jnp.float32))
        acc = matsum(masked)
        maskf = jnp.concatenate(mask_cols, axis=1)
        acc = acc + jax.lax.dot_general(maskf, b_ref[...],
                                        (((1,), (0,)), ((), ())),
                                        preferred_element_type=jnp.float32)
        out_ref[...] = acc

    for k in range(K):
        tail_ref[k] = curs[k][B - (K - 1):, :]


def kernel(weights, bias, input_0, input_1, input_2, input_3, input_4,
           input_5, input_6, input_7, input_8, input_9, input_10, input_11,
           input_12, input_13, input_14, input_15, input_16):
    ins = (input_0, input_1, input_2, input_3, input_4, input_5, input_6,
           input_7, input_8, input_9, input_10, input_11, input_12, input_13,
           input_14, input_15, input_16)
    n_out = N + K - 1
    grid = (pl.cdiv(n_out, B),)
    return pl.pallas_call(
        _body,
        grid=grid,
        in_specs=[
            pl.BlockSpec((K, OC, IC), lambda i: (0, 0, 0)),
            pl.BlockSpec((K, OC), lambda i: (0, 0)),
        ] + [pl.BlockSpec((B, IC), lambda i: (i, 0))] * K,
        out_specs=pl.BlockSpec((B, OC), lambda i: (i, 0)),
        out_shape=jax.ShapeDtypeStruct((n_out, OC), jnp.float32),
        scratch_shapes=[pltpu.VMEM((K, K - 1, IC), jnp.float32)],
        compiler_params=pltpu.CompilerParams(
            dimension_semantics=("arbitrary",)),
    )(weights, bias, *ins)


# speedup vs baseline: 9.7893x; 1.0199x over previous
import jax
import jax.numpy as jnp
from jax.experimental import pallas as pl
from jax.experimental.pallas import tpu as pltpu

K = 17
N = 50000
IC = 32
OC = 32
B = 1024
G = (N + K - 1 + B - 1) // B
NSLOT = 3


def _copy(in_refs, xbuf, sems, slot, kind, bi):
    copies = []
    for k in range(K):
        if kind == 'first':
            src = in_refs[k].at[pl.ds(0, B - k)]
            dst = xbuf.at[slot, k, pl.ds(k, B - k), :]
        elif kind == 'last':
            s = (G - 1) * B - k
            L = N - s
            src = in_refs[k].at[pl.ds(s, L)]
            dst = xbuf.at[slot, k, pl.ds(0, L), :]
        else:
            s = bi * B - k
            src = in_refs[k].at[pl.ds(s, B)]
            dst = xbuf.at[slot, k]
        copies.append(pltpu.make_async_copy(src, dst, sems.at[slot, k]))
    return copies


def _body(w_ref, b_ref, *refs):
    in_refs = refs[:K]
    out_ref = refs[K]
    xbuf = refs[K + 1]
    sems = refs[K + 2]

    i = pl.program_id(0)
    slot = jax.lax.rem(i, NSLOT)
    nslot = jax.lax.rem(i + 1, NSLOT)

    @pl.when(i == 0)
    def _prologue():
        for c in _copy(in_refs, xbuf, sems, 0, 'first', 0):
            c.start()

    @pl.when(i < G - 2)
    def _prefetch_interior():
        for c in _copy(in_refs, xbuf, sems, nslot, 'interior', i + 1):
            c.start()

    @pl.when(i == G - 2)
    def _prefetch_last():
        for c in _copy(in_refs, xbuf, sems, nslot, 'last', G - 1):
            c.start()

    @pl.when(i == 0)
    def _wait_first():
        for c in _copy(in_refs, xbuf, sems, slot, 'first', 0):
            c.wait()

    @pl.when(jnp.logical_and(i > 0, i < G - 1))
    def _wait_interior():
        for c in _copy(in_refs, xbuf, sems, slot, 'interior', i):
            c.wait()

    @pl.when(i == G - 1)
    def _wait_last():
        for c in _copy(in_refs, xbuf, sems, slot, 'last', G - 1):
            c.wait()

    def matsum(parts):
        acc = None
        for k in range(K):
            p = jax.lax.dot_general(
                parts[k], w_ref[k], (((1,), (1,)), ((), ())),
                preferred_element_type=jnp.float32)
            acc = p if acc is None else acc + p
        return acc

    @pl.when(jnp.logical_and(i > 0, i < G - 1))
    def _fast():
        acc = matsum([xbuf[slot, k] for k in range(K)])
        out_ref[...] = acc + jnp.sum(b_ref[...], axis=0, keepdims=True)

    @pl.when(jnp.logical_or(i == 0, i == G - 1))
    def _edge():
        m1 = jax.lax.broadcasted_iota(jnp.int32, (B, 1), 0) + i * B
        masked = []
        mask_cols = []
        for k in range(K):
            valid = jnp.logical_and(m1 >= k, m1 <= (N - 1) + k)
            masked.append(jnp.where(valid, xbuf[slot, k], 0.0))
            mask_cols.append(valid.astype(jnp.float32))
        acc = matsum(masked)
        maskf = jnp.concatenate(mask_cols, axis=1)
        out_ref[...] = acc + jax.lax.dot_general(
            maskf, b_ref[...], (((1,), (0,)), ((), ())),
            preferred_element_type=jnp.float32)


def kernel(weights, bias, input_0, input_1, input_2, input_3, input_4,
           input_5, input_6, input_7, input_8, input_9, input_10, input_11,
           input_12, input_13, input_14, input_15, input_16):
    ins = (input_0, input_1, input_2, input_3, input_4, input_5, input_6,
           input_7, input_8, input_9, input_10, input_11, input_12, input_13,
           input_14, input_15, input_16)
    n_out = N + K - 1
    return pl.pallas_call(
        _body,
        grid=(G,),
        in_specs=[
            pl.BlockSpec((K, OC, IC), lambda i: (0, 0, 0)),
            pl.BlockSpec((K, OC), lambda i: (0, 0)),
        ] + [pl.BlockSpec(memory_space=pl.ANY)] * K,
        out_specs=pl.BlockSpec((B, OC), lambda i: (i, 0)),
        out_shape=jax.ShapeDtypeStruct((n_out, OC), jnp.float32),
        scratch_shapes=[
            pltpu.VMEM((NSLOT, K, B, IC), jnp.float32),
            pltpu.SemaphoreType.DMA((NSLOT, K)),
        ],
        compiler_params=pltpu.CompilerParams(
            dimension_semantics=("arbitrary",)),
    )(weights, bias, *ins)
